# Initial kernel scaffold; baseline (speedup 1.0000x reference)
import jax
import jax.numpy as jnp
from jax.experimental import pallas as pl


def kernel(x_pruner, attention_weight, pre_edge_prob, pair_cand_idx, pair_child_idx, child_src_node, W1, b1, W2, b2, Ws, bs):
    raise NotImplementedError("write your pallas kernel here")


# speedup vs baseline: 150.1617x; 150.1617x over previous
import functools

import jax
import jax.numpy as jnp
from jax import lax
from jax.experimental import pallas as pl
from jax.experimental.pallas import tpu as pltpu
from jax.experimental.pallas import tpu_sc as plsc

EPS = 1e-6
NC, NS, L = 2, 16, 16
NW = NC * NS

M = 100000
MPAD = 100352
SLICE = MPAD // NS
D = 128
KE = 1600000
P = 6400000
RK = KE // 128
RP = P // 128

K_FULL = RK // NW
K_REM = RK - K_FULL * NW
K_WROWS = 15
K_NWIN = K_FULL // K_WROWS

P_FULL = RP // NW
P_WREM = RP - P_FULL * NW
P_WROWS = 22
P_NWIN = P_FULL // P_WROWS

_mesh = plsc.VectorSubcoreMesh(core_axis_name="c", subcore_axis_name="s")


def _worker_id():
    return lax.axis_index("s") * NC + lax.axis_index("c")


def _zero_vmem(buf, n):
    def body(i, _):
        buf[pl.ds(i * L, L)] = jnp.zeros((L,), jnp.float32)
        return 0
    lax.fori_loop(0, n // L, body, 0)


@functools.partial(
    pl.kernel,
    out_type=jax.ShapeDtypeStruct((2 * MPAD,), jnp.float32),
    mesh=_mesh,
    scratch_types=[
        pltpu.VMEM((K_WROWS * 128,), jnp.int32),
        pltpu.VMEM((K_WROWS * 128,), jnp.float32),
        pltpu.VMEM((SLICE,), jnp.float32),
        pltpu.VMEM_SHARED((MPAD,), jnp.float32),
    ],
)
def _segsum(z_hbm, src_hbm, out_hbm, idx_v, val_v, stage_v, ss_sh):
    c = lax.axis_index("c")
    s = lax.axis_index("s")
    wid = _worker_id()
    _zero_vmem(stage_v, SLICE)
    pltpu.sync_copy(stage_v, ss_sh.at[pl.ds(s * SLICE, SLICE)])
    plsc.subcore_barrier()

    base = wid * K_FULL + jnp.minimum(wid, K_REM)

    def do_rows(r0, nrows):
        n = nrows * 128
        pltpu.sync_copy(src_hbm.at[pl.ds(r0 * 128, n)], idx_v.at[pl.ds(0, n)])
        pltpu.sync_copy(z_hbm.at[pl.ds(r0 * 128, n)], val_v.at[pl.ds(0, n)])

        def eb(i, _):
            val_v[pl.ds(i * L, L)] = jnp.exp(val_v[pl.ds(i * L, L)])
            return 0
        lax.fori_loop(0, n // L, eb, 0)

        def rb(j, _):
            pltpu.sync_copy(val_v.at[pl.ds(j * 128, 128)],
                            ss_sh.at[idx_v.at[pl.ds(j * 128, 128)]], add=True)
            return 0
        lax.fori_loop(0, nrows, rb, 0)

    def win(g, _):
        do_rows(base + g * K_WROWS, K_WROWS)
        return 0
    lax.fori_loop(0, K_NWIN, win, 0)

    @pl.when(wid < K_REM)
    def _extra():
        do_rows(base + K_FULL, 1)

    plsc.subcore_barrier()
    pltpu.sync_copy(ss_sh.at[pl.ds(s * SLICE, SLICE)], stage_v)
    pltpu.sync_copy(stage_v, out_hbm.at[pl.ds(c * MPAD + s * SLICE, SLICE)])


@functools.partial(
    pl.kernel,
    out_type=jax.ShapeDtypeStruct((KE,), jnp.float32),
    mesh=_mesh,
    scratch_types=[
        pltpu.VMEM((K_WROWS * 128,), jnp.int32),
        pltpu.VMEM((K_WROWS * 128,), jnp.float32),
        pltpu.VMEM((K_WROWS * 128,), jnp.float32),
        pltpu.VMEM((SLICE,), jnp.float32),
        pltpu.VMEM((SLICE,), jnp.float32),
        pltpu.VMEM_SHARED((MPAD,), jnp.float32),
    ],
)
def _qkernel(z_hbm, src_hbm, ss_hbm, q_hbm, idx_v, val_v, ssb_v, a_v, b_v, ss_sh):
    c = lax.axis_index("c")
    s = lax.axis_index("s")
    wid = _worker_id()
    pltpu.sync_copy(ss_hbm.at[pl.ds(s * SLICE, SLICE)], a_v)
    pltpu.sync_copy(ss_hbm.at[pl.ds(MPAD + s * SLICE, SLICE)], b_v)

    def cb(i, _):
        a_v[pl.ds(i * L, L)] = a_v[pl.ds(i * L, L)] + b_v[pl.ds(i * L, L)] + EPS
        return 0
    lax.fori_loop(0, SLICE // L, cb, 0)
    pltpu.sync_copy(a_v, ss_sh.at[pl.ds(s * SLICE, SLICE)])
    plsc.subcore_barrier()

    base = wid * K_FULL + jnp.minimum(wid, K_REM)

    def do_rows(r0, nrows):
        n = nrows * 128
        pltpu.sync_copy(src_hbm.at[pl.ds(r0 * 128, n)], idx_v.at[pl.ds(0, n)])
        pltpu.sync_copy(z_hbm.at[pl.ds(r0 * 128, n)], val_v.at[pl.ds(0, n)])

        def gb(j, _):
            pltpu.sync_copy(ss_sh.at[idx_v.at[pl.ds(j * 128, 128)]],
                            ssb_v.at[pl.ds(j * 128, 128)])
            return 0
        lax.fori_loop(0, nrows, gb, 0)

        def qb(i, _):
            q = jnp.exp(val_v[pl.ds(i * L, L)]) / ssb_v[pl.ds(i * L, L)]
            val_v[pl.ds(i * L, L)] = jnp.clip(q, EPS, 1.0 - EPS)
            return 0
        lax.fori_loop(0, n // L, qb, 0)
        pltpu.sync_copy(val_v.at[pl.ds(0, n)], q_hbm.at[pl.ds(r0 * 128, n)])

    def win(g, _):
        do_rows(base + g * K_WROWS, K_WROWS)
        return 0
    lax.fori_loop(0, K_NWIN, win, 0)

    @pl.when(wid < K_REM)
    def _extra():
        do_rows(base + K_FULL, 1)


@functools.partial(
    pl.kernel,
    out_type=jax.ShapeDtypeStruct((2 * MPAD,), jnp.float32),
    mesh=_mesh,
    scratch_types=[
        pltpu.VMEM((P_WROWS * 128,), jnp.int32),
        pltpu.VMEM((P_WROWS * 128,), jnp.int32),
        pltpu.VMEM((P_WROWS * 128,), jnp.float32),
        pltpu.VMEM((P_WROWS * 128,), jnp.float32),
        pltpu.VMEM((10000,), jnp.float32),
        pltpu.VMEM_SHARED((KE,), jnp.float32),
        pltpu.VMEM_SHARED((MPAD,), jnp.float32),
    ],
)
def _aggregate(aw_hbm, cand_hbm, child_hbm, q_hbm, out_hbm,
               cnd_v, chd_v, aw_v, q_v, stage_v, q_sh, agg_sh):
    c = lax.axis_index("c")
    s = lax.axis_index("s")
    wid = _worker_id()
    _zero_vmem(stage_v, SLICE)
    pltpu.sync_copy(stage_v.at[pl.ds(0, SLICE)],
                    agg_sh.at[pl.ds(s * SLICE, SLICE)])
    def st(k, _):
        off = s * (KE // NS) + k * 10000
        pltpu.sync_copy(q_hbm.at[pl.ds(off, 10000)], stage_v)
        pltpu.sync_copy(stage_v, q_sh.at[pl.ds(off, 10000)])
        return 0
    lax.fori_loop(0, KE // NS // 10000, st, 0)
    plsc.subcore_barrier()

    base = wid * P_FULL + jnp.minimum(wid, P_WREM)

    def do_rows(r0, nrows):
        n = nrows * 128
        pltpu.sync_copy(cand_hbm.at[pl.ds(r0 * 128, n)], cnd_v.at[pl.ds(0, n)])
        pltpu.sync_copy(child_hbm.at[pl.ds(r0 * 128, n)], chd_v.at[pl.ds(0, n)])
        pltpu.sync_copy(aw_hbm.at[pl.ds(r0 * 128, n)], aw_v.at[pl.ds(0, n)])

        def gb(j, _):
            pltpu.sync_copy(q_sh.at[chd_v.at[pl.ds(j * 128, 128)]],
                            q_v.at[pl.ds(j * 128, 128)])
            return 0
        lax.fori_loop(0, nrows, gb, 0)

        def mb(i, _):
            aw_v[pl.ds(i * L, L)] = aw_v[pl.ds(i * L, L)] * q_v[pl.ds(i * L, L)]
            return 0
        lax.fori_loop(0, n // L, mb, 0)

        def sb(j, _):
            pltpu.sync_copy(aw_v.at[pl.ds(j * 128, 128)],
                            agg_sh.at[cnd_v.at[pl.ds(j * 128, 128)]], add=True)
            return 0
        lax.fori_loop(0, nrows, sb, 0)

    def win(g, _):
        do_rows(base + g * P_WROWS, P_WROWS)
        return 0
    lax.fori_loop(0, P_NWIN, win, 0)

    @pl.when(wid < P_WREM)
    def _extra():
        do_rows(base + P_FULL, 1)

    plsc.subcore_barrier()
    pltpu.sync_copy(agg_sh.at[pl.ds(s * SLICE, SLICE)], stage_v.at[pl.ds(0, SLICE)])
    pltpu.sync_copy(stage_v.at[pl.ds(0, SLICE)],
                    out_hbm.at[pl.ds(c * MPAD + s * SLICE, SLICE)])


BM = 1024
GRID = MPAD // BM


def _mlp_body(x_ref, w1_ref, b1_ref, w2_ref, b2_ref, ws_ref, bs_ref, o_ref):
    x = x_ref[...]
    h = lax.dot_general(x, w1_ref[...], (((1,), (1,)), ((), ())),
                        preferred_element_type=jnp.float32) + b1_ref[...]
    h = jnp.maximum(h, 0.0)
    h = lax.dot_general(h, w2_ref[...], (((1,), (1,)), ((), ())),
                        preferred_element_type=jnp.float32) + b2_ref[...]
    o_ref[...] = jnp.sum(h * ws_ref[...], axis=1) + bs_ref[...]


def _mlp(x, W1, b1, W2, b2, Ws, bs):
    return pl.pallas_call(
        _mlp_body,
        grid=(GRID,),
        in_specs=[
            pl.BlockSpec((BM, D), lambda i: (i, 0)),
            pl.BlockSpec((D, D), lambda i: (0, 0)),
            pl.BlockSpec((D,), lambda i: (0,)),
            pl.BlockSpec((D, D), lambda i: (0, 0)),
            pl.BlockSpec((D,), lambda i: (0,)),
            pl.BlockSpec((1, D), lambda i: (0, 0)),
            pl.BlockSpec((1,), lambda i: (0,)),
        ],
        out_specs=pl.BlockSpec((BM,), lambda i: (i,)),
        out_shape=jax.ShapeDtypeStruct((MPAD,), jnp.float32),
    )(x, W1, b1, W2, b2, Ws, bs)


def _fin_body(lg_ref, a0_ref, a1_ref, s_ref, p_ref):
    lg = lg_ref[...]
    agg = jnp.clip(a0_ref[...] + a1_ref[...], EPS, 1.0 - EPS)
    ctx = jnp.log(agg) - jnp.log1p(-agg)
    s_ref[...] = jax.nn.sigmoid(lg)
    p_ref[...] = lg + ctx


def _finalize(logits, a0, a1):
    return pl.pallas_call(
        _fin_body,
        grid=(GRID,),
        in_specs=[pl.BlockSpec((BM,), lambda i: (i,))] * 3,
        out_specs=[pl.BlockSpec((BM,), lambda i: (i,))] * 2,
        out_shape=[jax.ShapeDtypeStruct((MPAD,), jnp.float32)] * 2,
    )(logits, a0, a1)


def kernel(x_pruner, attention_weight, pre_edge_prob, pair_cand_idx,
           pair_child_idx, child_src_node, W1, b1, W2, b2, Ws, bs):
    src = child_src_node.astype(jnp.int32)
    cand = pair_cand_idx.astype(jnp.int32)
    child = pair_child_idx.astype(jnp.int32)

    logits = _mlp(x_pruner, W1, b1, W2, b2, Ws, bs)
    ss = _segsum(pre_edge_prob, src)
    q = _qkernel(pre_edge_prob, src, ss)
    aggs = _aggregate(attention_weight, cand, child, q)
    s_full, p_full = _finalize(logits, aggs[:MPAD], aggs[MPAD:])
    return s_full[:M], p_full[:M]
